# baseline (speedup 1.0000x reference)
import functools

import jax
import jax.numpy as jnp
from jax import lax
from jax.experimental import pallas as pl
from jax.experimental.pallas import tpu as pltpu
from jax.experimental.pallas import tpu_sc as plsc

_B = 1024
_NM = 200
_EMB = 128
_MAX_CHG = 2000
_N = _B * _NM
_CHG_ROWS = 2 * _MAX_CHG + 1
_TIME_ROWS = 14400 // 5 + 1

_SC_CORES = 2
_SC_SUBCORES = 16
_NW = _SC_CORES * _SC_SUBCORES
_GK = 400
_NH = 1
_BH = _B // _NH


def _idx_body(c3_ref, c4_ref, chg_ref, time_ref):
    c3 = c3_ref[...]
    c4 = c4_ref[...]
    d3 = c3 - c3[:, 0:1]
    d4 = c4 - c4[:, 0:1]
    b = lax.broadcasted_iota(jnp.int32, (_B, 1), 0)
    rep_off = ((b % _BH) // (_BH // 16)) * (_CHG_ROWS + _TIME_ROWS)
    chg_ref[...] = rep_off + jnp.clip(d3, -_MAX_CHG, _MAX_CHG) + _MAX_CHG
    time_ref[...] = rep_off + _CHG_ROWS + d4 // 5


def _idx_call(c3, c4):
    return pl.pallas_call(
        _idx_body,
        out_shape=[
            jax.ShapeDtypeStruct((_B, _NM), jnp.int32),
            jax.ShapeDtypeStruct((_B, _NM), jnp.int32),
        ],
    )(c3, c4)


def _sc_gather_call(table, gidx):
    total = gidx.shape[0]
    per_w = total // _NW
    nch = per_w // _GK
    assert nch % 2 == 0
    mesh = plsc.VectorSubcoreMesh(core_axis_name="c", subcore_axis_name="s")

    @functools.partial(
        pl.kernel,
        mesh=mesh,
        out_type=jax.ShapeDtypeStruct((total, _EMB), jnp.float32),
        scratch_types=[
            pltpu.VMEM((per_w,), jnp.int32),
            pltpu.VMEM((_GK, _EMB), jnp.float32),
            pltpu.VMEM((_GK, _EMB), jnp.float32),
            pltpu.SemaphoreType.DMA,
            pltpu.SemaphoreType.DMA,
            pltpu.SemaphoreType.DMA,
            pltpu.SemaphoreType.DMA,
        ],
    )
    def k(table_hbm, idx_hbm, out_hbm, idx_v, r0, r1, g0, g1, w0, w1):
        sid = lax.axis_index("s")
        wid = sid * _SC_CORES + lax.axis_index("c")
        base = wid * per_w
        pltpu.sync_copy(idx_hbm.at[pl.ds(base, per_w)], idx_v)

        def gather(c, buf, sem):
            return pltpu.make_async_copy(
                table_hbm.at[idx_v.at[pl.ds(c * _GK, _GK)]], buf, sem)

        def writeout(c, buf, sem):
            return pltpu.make_async_copy(
                buf, out_hbm.at[pl.ds(base + c * _GK, _GK)], sem)

        gather(0, r0, g0).start()
        gather(1, r1, g1).start()

        @pl.loop(0, nch // 2 - 1)
        def _(p):
            c0 = 2 * p
            gather(c0, r0, g0).wait()
            writeout(c0, r0, w0).start()
            gather(c0 + 1, r1, g1).wait()
            writeout(c0 + 1, r1, w1).start()
            writeout(c0, r0, w0).wait()
            gather(c0 + 2, r0, g0).start()
            writeout(c0 + 1, r1, w1).wait()
            gather(c0 + 3, r1, g1).start()

        c0 = nch - 2
        gather(c0, r0, g0).wait()
        writeout(c0, r0, w0).start()
        gather(c0 + 1, r1, g1).wait()
        writeout(c0 + 1, r1, w1).start()
        writeout(c0, r0, w0).wait()
        writeout(c0 + 1, r1, w1).wait()

    return k(table, gidx)


def _combine_body(f15_ref, g1_ref, g2_ref, ts_ref, wot_ref,
                  wlob_ref, blob_ref, gam_ref, bet_ref, out_ref):
    oi = f15_ref[:, 0:1]
    pl_i = (oi >> 13) & 127
    pv_i = (oi >> 6) & 127
    in_i = oi & 63
    j = lax.broadcasted_iota(jnp.int32, (1, 320), 1)
    oh = ((j == pl_i) | (j == 128 + pv_i) | (j == 256 + in_i))
    small = jnp.dot(oh.astype(jnp.bfloat16), ts_ref[...],
                    preferred_element_type=jnp.float32)
    ot = oi >> 20
    ot_tok = jnp.where(ot == 0, wot_ref[0:1, :],
                       jnp.where(ot == 1, wot_ref[1:2, :], wot_ref[2:3, :]))
    lob = f15_ref[:, 5:15].astype(jnp.float32)
    x = jnp.dot(lob, wlob_ref[...],
                preferred_element_type=jnp.float32) + blob_ref[...]
    mu = jnp.mean(x, axis=1, keepdims=True)
    xc = x - mu
    var = jnp.mean(xc * xc, axis=1, keepdims=True)
    ln = xc / jnp.sqrt(var + 1e-5) * gam_ref[...] + bet_ref[...]
    gsum = g1_ref[...].astype(jnp.float32) + g2_ref[...].astype(jnp.float32)
    out_ref[...] = small + ot_tok + gsum + ln


def _combine_call(f15, gath, ts, wot, wlob, blob, gam, bet, rows_per_step=1024):
    n_steps = f15.shape[0] // rows_per_step
    r = rows_per_step
    return pl.pallas_call(
        _combine_body,
        grid=(n_steps,),
        in_specs=[
            pl.BlockSpec((r, 15), lambda i: (i, 0)),
            pl.BlockSpec((r, _EMB), lambda i: (i, 0)),
            pl.BlockSpec((r, _EMB), lambda i, _n=n_steps: (i + _n, 0)),
            pl.BlockSpec((320, _EMB), lambda i: (0, 0)),
            pl.BlockSpec((3, _EMB), lambda i: (0, 0)),
            pl.BlockSpec((10, _EMB), lambda i: (0, 0)),
            pl.BlockSpec((1, _EMB), lambda i: (0, 0)),
            pl.BlockSpec((1, _EMB), lambda i: (0, 0)),
            pl.BlockSpec((1, _EMB), lambda i: (0, 0)),
        ],
        out_specs=pl.BlockSpec((r, _EMB), lambda i: (i, 0)),
        out_shape=jax.ShapeDtypeStruct((f15.shape[0], _EMB), jnp.float32),
        compiler_params=pltpu.CompilerParams(
            dimension_semantics=("parallel",)),
    )(f15, gath, gath, ts, wot, wlob, blob, gam, bet)


def kernel(features, W_order_type, W_price_level, W_pred_order_volume,
           W_order_interval, W_chg_to_open, W_time_to_open, W_lob, b_lob,
           ln_gamma, ln_beta):
    X = features.reshape(_B, _NM, 15)
    c3 = X[:, :, 3]
    c4 = X[:, :, 4]

    chg_gidx, time_gidx = _idx_call(c3, c4)
    table = jnp.tile(jnp.concatenate([W_chg_to_open, W_time_to_open],
                                     axis=0), (16, 1))
    ts = jnp.concatenate([W_price_level, W_pred_order_volume,
                          W_order_interval], axis=0).astype(jnp.bfloat16)

    nh = _BH * _NM
    outs = []
    for h in range(_NH):
        sl = slice(h * _BH, (h + 1) * _BH)
        gidx = jnp.concatenate([chg_gidx[sl].reshape(-1),
                                time_gidx[sl].reshape(-1)])
        gath = _sc_gather_call(table, gidx)
        f15 = X[sl].reshape(nh, 15)
        outs.append(_combine_call(
            f15, gath, ts, W_order_type, W_lob,
            b_lob.reshape(1, _EMB), ln_gamma.reshape(1, _EMB),
            ln_beta.reshape(1, _EMB)))
    out = jnp.concatenate(outs, axis=0)
    return out.reshape(_B, _NM * _EMB)


# speedup vs baseline: 1.0662x; 1.0662x over previous
import functools

import jax
import jax.numpy as jnp
from jax import lax
from jax.experimental import pallas as pl
from jax.experimental.pallas import tpu as pltpu
from jax.experimental.pallas import tpu_sc as plsc

_B = 1024
_NM = 200
_EMB = 128
_MAX_CHG = 2000
_N = _B * _NM
_CHG_ROWS = 2 * _MAX_CHG + 1
_TIME_ROWS = 14400 // 5 + 1

_SC_CORES = 2
_SC_SUBCORES = 16
_NW = _SC_CORES * _SC_SUBCORES
_GK = 256
_NH = 1
_BH = _B // _NH


def _idx_body(c3_ref, c4_ref, chg_ref, time_ref):
    c3 = c3_ref[...]
    c4 = c4_ref[...]
    d3 = c3 - c3[:, 0:1]
    d4 = c4 - c4[:, 0:1]
    b = lax.broadcasted_iota(jnp.int32, (_B, 1), 0)
    rep_off = ((b % _BH) // (_BH // 16)) * (_CHG_ROWS + _TIME_ROWS)
    chg_ref[...] = rep_off + jnp.clip(d3, -_MAX_CHG, _MAX_CHG) + _MAX_CHG
    time_ref[...] = rep_off + _CHG_ROWS + d4 // 5


def _idx_call(c3, c4):
    return pl.pallas_call(
        _idx_body,
        out_shape=[
            jax.ShapeDtypeStruct((_B, _NM), jnp.int32),
            jax.ShapeDtypeStruct((_B, _NM), jnp.int32),
        ],
    )(c3, c4)


def _sc_gather_call(table, gidx):
    total = gidx.shape[0]
    per_w = total // _NW
    nch = per_w // _GK
    assert nch % 2 == 0
    mesh = plsc.VectorSubcoreMesh(core_axis_name="c", subcore_axis_name="s")

    @functools.partial(
        pl.kernel,
        mesh=mesh,
        out_type=jax.ShapeDtypeStruct((total, _EMB), jnp.float32),
        scratch_types=[
            pltpu.VMEM((per_w,), jnp.int32),
            pltpu.VMEM((_GK, _EMB), jnp.float32),
            pltpu.VMEM((_GK, _EMB), jnp.float32),
            pltpu.SemaphoreType.DMA,
            pltpu.SemaphoreType.DMA,
            pltpu.SemaphoreType.DMA,
            pltpu.SemaphoreType.DMA,
        ],
    )
    def k(table_hbm, idx_hbm, out_hbm, idx_v, r0, r1, g0, g1, w0, w1):
        sid = lax.axis_index("s")
        wid = sid * _SC_CORES + lax.axis_index("c")
        base = wid * per_w
        pltpu.sync_copy(idx_hbm.at[pl.ds(base, per_w)], idx_v)

        def gather(c, buf, sem):
            return pltpu.make_async_copy(
                table_hbm.at[idx_v.at[pl.ds(c * _GK, _GK)]], buf, sem)

        def writeout(c, buf, sem):
            return pltpu.make_async_copy(
                buf, out_hbm.at[pl.ds(base + c * _GK, _GK)], sem)

        gather(0, r0, g0).start()
        gather(1, r1, g1).start()

        @pl.loop(0, nch // 2 - 1)
        def _(p):
            c0 = 2 * p
            gather(c0, r0, g0).wait()
            writeout(c0, r0, w0).start()
            gather(c0 + 1, r1, g1).wait()
            writeout(c0 + 1, r1, w1).start()
            writeout(c0, r0, w0).wait()
            gather(c0 + 2, r0, g0).start()
            writeout(c0 + 1, r1, w1).wait()
            gather(c0 + 3, r1, g1).start()

        c0 = nch - 2
        gather(c0, r0, g0).wait()
        writeout(c0, r0, w0).start()
        gather(c0 + 1, r1, g1).wait()
        writeout(c0 + 1, r1, w1).start()
        writeout(c0, r0, w0).wait()
        writeout(c0 + 1, r1, w1).wait()

    return k(table, gidx)


def _combine_body(oi_ref, lob_ref, g1_ref, g2_ref, ts_ref, wot_ref,
                  wlob_ref, blob_ref, gam_ref, bet_ref, out_ref):
    oi = oi_ref[...]
    pl_i = (oi >> 13) & 127
    pv_i = (oi >> 6) & 127
    in_i = oi & 63
    j = lax.broadcasted_iota(jnp.int32, (1, 320), 1)
    oh = ((j == pl_i) | (j == 128 + pv_i) | (j == 256 + in_i))
    small = jnp.dot(oh.astype(jnp.float32), ts_ref[...],
                    preferred_element_type=jnp.float32)
    ot = oi >> 20
    ot_tok = jnp.where(ot == 0, wot_ref[0:1, :],
                       jnp.where(ot == 1, wot_ref[1:2, :], wot_ref[2:3, :]))
    x = jnp.dot(lob_ref[...], wlob_ref[...],
                preferred_element_type=jnp.float32) + blob_ref[...]
    mu = jnp.mean(x, axis=1, keepdims=True)
    xc = x - mu
    var = jnp.mean(xc * xc, axis=1, keepdims=True)
    ln = xc / jnp.sqrt(var + 1e-5) * gam_ref[...] + bet_ref[...]
    gsum = g1_ref[...].astype(jnp.float32) + g2_ref[...].astype(jnp.float32)
    out_ref[...] = small + ot_tok + gsum + ln


def _combine_call(oi, lob, gath, ts, wot, wlob, blob, gam, bet, rows_per_step=1024):
    n_steps = oi.shape[0] // rows_per_step
    r = rows_per_step
    return pl.pallas_call(
        _combine_body,
        grid=(n_steps,),
        in_specs=[
            pl.BlockSpec((r, 1), lambda i: (i, 0)),
            pl.BlockSpec((r, 10), lambda i: (i, 0)),
            pl.BlockSpec((r, _EMB), lambda i: (i, 0)),
            pl.BlockSpec((r, _EMB), lambda i, _n=n_steps: (i + _n, 0)),
            pl.BlockSpec((320, _EMB), lambda i: (0, 0)),
            pl.BlockSpec((3, _EMB), lambda i: (0, 0)),
            pl.BlockSpec((10, _EMB), lambda i: (0, 0)),
            pl.BlockSpec((1, _EMB), lambda i: (0, 0)),
            pl.BlockSpec((1, _EMB), lambda i: (0, 0)),
            pl.BlockSpec((1, _EMB), lambda i: (0, 0)),
        ],
        out_specs=pl.BlockSpec((r, _EMB), lambda i: (i, 0)),
        out_shape=jax.ShapeDtypeStruct((oi.shape[0], _EMB), jnp.float32),
        compiler_params=pltpu.CompilerParams(
            dimension_semantics=("parallel",)),
    )(oi, lob, gath, gath, ts, wot, wlob, blob, gam, bet)


def kernel(features, W_order_type, W_price_level, W_pred_order_volume,
           W_order_interval, W_chg_to_open, W_time_to_open, W_lob, b_lob,
           ln_gamma, ln_beta):
    X = features.reshape(_B, _NM, 15)
    c3 = X[:, :, 3]
    c4 = X[:, :, 4]

    chg_gidx, time_gidx = _idx_call(c3, c4)
    table = jnp.tile(jnp.concatenate([W_chg_to_open, W_time_to_open],
                                     axis=0), (16, 1))
    ts = jnp.concatenate([W_price_level, W_pred_order_volume,
                          W_order_interval], axis=0)

    nh = _BH * _NM
    outs = []
    for h in range(_NH):
        sl = slice(h * _BH, (h + 1) * _BH)
        gidx = jnp.concatenate([chg_gidx[sl].reshape(-1),
                                time_gidx[sl].reshape(-1)])
        gath = _sc_gather_call(table, gidx)
        oi = X[sl, :, 0].reshape(nh, 1)
        lob = X[sl, :, 5:15].reshape(nh, 10).astype(jnp.float32)
        outs.append(_combine_call(
            oi, lob, gath, ts, W_order_type, W_lob,
            b_lob.reshape(1, _EMB), ln_gamma.reshape(1, _EMB),
            ln_beta.reshape(1, _EMB)))
    out = jnp.concatenate(outs, axis=0)
    return out.reshape(_B, _NM * _EMB)
